# Initial kernel scaffold; baseline (speedup 1.0000x reference)
import jax
import jax.numpy as jnp
from jax.experimental import pallas as pl


def kernel(x, edge_index, W1, b1, g1, be1, W2, b2, Wf, bf):
    raise NotImplementedError("write your pallas kernel here")


# speedup vs baseline: 9.0572x; 9.0572x over previous
import functools

import jax
import jax.numpy as jnp
from jax import lax
from jax.experimental import pallas as pl
from jax.experimental.pallas import tpu as pltpu
from jax.experimental.pallas import tpu_sc as plsc

_NC = 2
_NS = 16
_NW = _NC * _NS
_CH = 128
_DEGW = 16


def _npad(n):
    blk = _NS * _CH
    return ((n + 1 + blk - 1) // blk) * blk


@functools.lru_cache(maxsize=None)
def _make_q(n, d, nch):
    npad = _npad(n)
    zch = (npad // _NS) // _CH
    orow = npad // _NS
    mesh = plsc.VectorSubcoreMesh(core_axis_name="c", subcore_axis_name="s",
                                  num_cores=_NC, num_subcores=_NS)

    @functools.partial(
        pl.kernel,
        out_type=jax.ShapeDtypeStruct((_NC, npad, d), jnp.float32),
        mesh=mesh,
        compiler_params=pltpu.CompilerParams(use_tc_tiling_on_sc=False),
        scratch_types=[
            pltpu.VMEM((nch, _CH), jnp.int32),
            pltpu.VMEM((nch, _CH), jnp.int32),
            pltpu.VMEM((_CH, d), jnp.float32),
            pltpu.VMEM((_CH, d), jnp.float32),
            pltpu.VMEM_SHARED((npad, d), jnp.float32),
            pltpu.SemaphoreType.DMA,
        ],
    )
    def qk(table, ridx, cidx, zrows, out, ridx_v, cidx_v, buf, zbuf, acc, sem):
        c = lax.axis_index("c")
        s = lax.axis_index("s")
        pltpu.sync_copy(zrows, zbuf)
        base = s * (npad // _NS)
        for j in range(zch):
            pltpu.sync_copy(zbuf, acc.at[pl.ds(base + j * _CH, _CH)])
        pltpu.sync_copy(ridx.at[c, s], ridx_v)
        pltpu.sync_copy(cidx.at[c, s], cidx_v)
        plsc.subcore_barrier()

        def chunk(j, carry):
            pltpu.async_copy(table.at[ridx_v.at[j]], buf, sem).wait()
            pltpu.sync_copy(buf, acc.at[cidx_v.at[j]], add=True)
            return carry

        lax.fori_loop(0, nch, chunk, 0)
        plsc.subcore_barrier()
        pltpu.sync_copy(acc.at[pl.ds(s * orow, orow)],
                        out.at[c, pl.ds(s * orow, orow)])

    return qk


@functools.lru_cache(maxsize=None)
def _make_deg(n, nch):
    npad = _npad(n)
    zch = (npad // _NS) // _CH
    orow = npad // _NS
    mesh = plsc.VectorSubcoreMesh(core_axis_name="c", subcore_axis_name="s",
                                  num_cores=_NC, num_subcores=_NS)

    @functools.partial(
        pl.kernel,
        out_type=jax.ShapeDtypeStruct((_NC, npad, _DEGW), jnp.float32),
        mesh=mesh,
        compiler_params=pltpu.CompilerParams(use_tc_tiling_on_sc=False),
        scratch_types=[
            pltpu.VMEM((nch, _CH), jnp.int32),
            pltpu.VMEM((_CH, _DEGW), jnp.float32),
            pltpu.VMEM((_CH, _DEGW), jnp.float32),
            pltpu.VMEM_SHARED((npad, _DEGW), jnp.float32),
        ],
    )
    def dk(ridx, orows, zrows, out, ridx_v, obuf, zbuf, acc):
        c = lax.axis_index("c")
        s = lax.axis_index("s")
        pltpu.sync_copy(zrows, zbuf)
        pltpu.sync_copy(orows, obuf)
        base = s * (npad // _NS)
        for j in range(zch):
            pltpu.sync_copy(zbuf, acc.at[pl.ds(base + j * _CH, _CH)])
        pltpu.sync_copy(ridx.at[c, s], ridx_v)
        plsc.subcore_barrier()

        def chunk(j, carry):
            pltpu.sync_copy(obuf, acc.at[ridx_v.at[j]], add=True)
            return carry

        lax.fori_loop(0, nch, chunk, 0)
        plsc.subcore_barrier()
        pltpu.sync_copy(acc.at[pl.ds(s * orow, orow)],
                        out.at[c, pl.ds(s * orow, orow)])

    return dk


def _dis_of(dg_ref, n):
    dg = dg_ref[...]
    deg = dg[0, :n] + dg[1, :n]
    dis = jnp.where(deg > 0, lax.rsqrt(jnp.maximum(deg, 1e-12)), 0.0)
    return dis[:, 0:1]


def _psum(p_ref, n):
    p = p_ref[...]
    return p[0, :n] + p[1, :n]


def _tc1_body(x_ref, w_ref, b_ref, dg_ref, sa1_ref, sa2_ref, st_ref):
    dh = st_ref.shape[1]
    dis = _dis_of(dg_ref, x_ref.shape[0])
    a = jnp.dot(x_ref[...], w_ref[...], preferred_element_type=jnp.float32)
    sa1_ref[...] = a[:, dh:2 * dh] * dis
    sa2_ref[...] = a[:, 2 * dh:] * dis
    st_ref[...] = a[:, :dh] - a[:, 2 * dh:] + b_ref[...]


def _tc3_body(g1a_ref, g1b_ref, dg_ref, st_ref, in2_ref, p1_ref):
    n = st_ref.shape[0]
    dis = _dis_of(dg_ref, n)
    in2_ref[...] = _psum(g1b_ref, n) * (dis * dis)
    p1_ref[...] = st_ref[...] - _psum(g1a_ref, n) * dis


def _tc5_body(g2_ref, dg_ref, p1_ref, g_ref, be_ref, h_ref, hs_ref):
    n = p1_ref.shape[0]
    dis = _dis_of(dg_ref, n)
    g2s = _psum(g2_ref, n)
    h1 = p1_ref[...] + 2.0 * dis * g2s
    m = jnp.mean(h1, axis=0, keepdims=True)
    v = jnp.mean((h1 - m) ** 2, axis=0, keepdims=True)
    hb = (h1 - m) * lax.rsqrt(v + 1e-5) * g_ref[...] + be_ref[...]
    h = jnp.where(hb >= 0, hb, 0.01 * hb)
    h_ref[...] = h
    hs_ref[...] = h * dis


def _tc7_body(q1_ref, dg_ref, h_ref, w2_ref, b2_ref, in3_ref, acc_ref):
    n = h_ref.shape[0]
    dis = _dis_of(dg_ref, n)
    q1s = _psum(q1_ref, n)
    tx1 = -dis * q1s
    in3_ref[...] = dis * tx1
    w2 = w2_ref[...]
    acc_ref[...] = (jnp.dot(h_ref[...], w2[0], preferred_element_type=jnp.float32)
                    + jnp.dot(tx1, w2[1], preferred_element_type=jnp.float32)
                    + b2_ref[...])


def _tc9_body(q2_ref, dg_ref, acc_ref, h_ref, w2_ref, wf_ref, bf_ref, out_ref):
    n = h_ref.shape[0]
    dis = _dis_of(dg_ref, n)
    q2s = _psum(q2_ref, n)
    tx2 = -2.0 * dis * q2s - h_ref[...]
    o = acc_ref[...] + jnp.dot(tx2, w2_ref[...][2],
                               preferred_element_type=jnp.float32)
    out_ref[...] = jnp.dot(o, wf_ref[...],
                           preferred_element_type=jnp.float32) + bf_ref[...]


def kernel(x, edge_index, W1, b1, g1, be1, W2, b2, Wf, bf):
    n, d_in = x.shape
    kc, _, dh = W1.shape
    assert kc == 3, "kernel specialized for K=3 Chebyshev order"
    e = edge_index.shape[1]
    nclass = Wf.shape[1]

    per_w = -(-e // _NW)
    nch = -(-per_w // _CH)
    epad = _NW * nch * _CH
    padn = epad - e

    row = edge_index[0]
    col = edge_index[1]
    ridx_g = jnp.concatenate(
        [row, jnp.zeros((padn,), jnp.int32)]).reshape(_NC, _NS, nch, _CH)
    ridx_d = jnp.concatenate(
        [row, jnp.full((padn,), n, jnp.int32)]).reshape(_NC, _NS, nch, _CH)
    cidx = jnp.concatenate(
        [col, jnp.full((padn,), n, jnp.int32)]).reshape(_NC, _NS, nch, _CH)

    zq64 = jnp.zeros((_CH, dh), jnp.float32)
    zdeg = jnp.zeros((_CH, _DEGW), jnp.float32)
    odeg = jnp.ones((_CH, _DEGW), jnp.float32)

    w1r = jnp.transpose(W1, (1, 0, 2)).reshape(d_in, kc * dh)

    degp = _make_deg(n, nch)(ridx_d, odeg, zdeg)

    sa1, sa2, stash = pl.pallas_call(
        _tc1_body,
        out_shape=(jax.ShapeDtypeStruct((n, dh), jnp.float32),
                   jax.ShapeDtypeStruct((n, dh), jnp.float32),
                   jax.ShapeDtypeStruct((n, dh), jnp.float32)),
    )(x, w1r, b1.reshape(1, dh), degp)

    g1pa = _make_q(n, dh, nch)(sa1, ridx_g, cidx, zq64)
    g1pb = _make_q(n, dh, nch)(sa2, ridx_g, cidx, zq64)

    in2, p1 = pl.pallas_call(
        _tc3_body,
        out_shape=(jax.ShapeDtypeStruct((n, dh), jnp.float32),
                   jax.ShapeDtypeStruct((n, dh), jnp.float32)),
    )(g1pa, g1pb, degp, stash)

    g2p = _make_q(n, dh, nch)(in2, ridx_g, cidx, zq64)

    h, hs = pl.pallas_call(
        _tc5_body,
        out_shape=(jax.ShapeDtypeStruct((n, dh), jnp.float32),
                   jax.ShapeDtypeStruct((n, dh), jnp.float32)),
    )(g2p, degp, p1, g1.reshape(1, dh), be1.reshape(1, dh))

    q1p = _make_q(n, dh, nch)(hs, ridx_g, cidx, zq64)

    in3, acc2 = pl.pallas_call(
        _tc7_body,
        out_shape=(jax.ShapeDtypeStruct((n, dh), jnp.float32),
                   jax.ShapeDtypeStruct((n, dh), jnp.float32)),
    )(q1p, degp, h, W2, b2.reshape(1, dh))

    q2p = _make_q(n, dh, nch)(in3, ridx_g, cidx, zq64)

    out = pl.pallas_call(
        _tc9_body,
        out_shape=jax.ShapeDtypeStruct((n, nclass), jnp.float32),
    )(q2p, degp, acc2, h, W2, Wf, bf.reshape(1, nclass))

    return out
